# Initial kernel scaffold; baseline (speedup 1.0000x reference)
import jax
import jax.numpy as jnp
from jax.experimental import pallas as pl


def kernel(temporal_data, x, edge_index, Wt1, bt1, Wt2, bt2, W1, as1, ad1, b1, W2, as2, ad2, b2, Wc1, bc1, Wc2, bc2):
    raise NotImplementedError("write your pallas kernel here")


# speedup vs baseline: 1.0007x; 1.0007x over previous
import functools

import jax
import jax.numpy as jnp
from jax.experimental import pallas as pl

N = 50000
T = 20


def _temporal_body(td_ref, wt1_ref, bt1_ref, wt2_ref, bt2_ref, o_ref):
    y = jnp.maximum(td_ref[...] @ wt1_ref[...] + bt1_ref[...][None, :], 0.0)
    b = y.shape[0] // T
    ym = y.reshape(b, T, 64).mean(axis=1)
    o_ref[...] = ym @ wt2_ref[...] + bt2_ref[...][None, :]


def _temporal_mlp(td, Wt1, bt1, Wt2, bt2):
    B = 2000
    td2 = td.reshape(N * T, 10)
    grid = (N // B,)
    return pl.pallas_call(
        _temporal_body,
        grid=grid,
        in_specs=[
            pl.BlockSpec((B * T, 10), lambda i: (i, 0)),
            pl.BlockSpec((10, 64), lambda i: (0, 0)),
            pl.BlockSpec((64,), lambda i: (0,)),
            pl.BlockSpec((64, 64), lambda i: (0, 0)),
            pl.BlockSpec((64,), lambda i: (0,)),
        ],
        out_specs=pl.BlockSpec((B, 64), lambda i: (i, 0)),
        out_shape=jax.ShapeDtypeStruct((N, 64), jnp.float32),
    )(td2, Wt1, bt1, Wt2, bt2)


def _head_body(t_ref, g_ref, wc1_ref, bc1_ref, wc2_ref, bc2_ref, o_ref):
    c = jnp.concatenate([t_ref[...], g_ref[...]], axis=-1)
    h = jnp.maximum(c @ wc1_ref[...] + bc1_ref[...][None, :], 0.0)
    o_ref[...] = jax.nn.sigmoid(h @ wc2_ref[...] + bc2_ref[...][None, :])


def _head_mlp(t_mean, g, Wc1, bc1, Wc2, bc2):
    B = 2000
    grid = (N // B,)
    return pl.pallas_call(
        _head_body,
        grid=grid,
        in_specs=[
            pl.BlockSpec((B, 64), lambda i: (i, 0)),
            pl.BlockSpec((B, 64), lambda i: (i, 0)),
            pl.BlockSpec((128, 64), lambda i: (0, 0)),
            pl.BlockSpec((64,), lambda i: (0,)),
            pl.BlockSpec((64, 1), lambda i: (0, 0)),
            pl.BlockSpec((1,), lambda i: (0,)),
        ],
        out_specs=pl.BlockSpec((B, 1), lambda i: (i, 0)),
        out_shape=jax.ShapeDtypeStruct((N, 1), jnp.float32),
    )(t_mean, g, Wc1, bc1, Wc2, bc2)


def _gat_conv(x, src, dst, W, att_src, att_dst, bias, heads, out_ch, concat):
    n = x.shape[0]
    h = (x @ W).reshape(n, heads, out_ch)
    a_src = (h * att_src[None, :, :]).sum(-1)
    a_dst = (h * att_dst[None, :, :]).sum(-1)
    e = a_src[src] + a_dst[dst]
    e = jax.nn.leaky_relu(e, 0.2)
    e_max = jax.ops.segment_max(e, dst, num_segments=n)
    e_max = jnp.where(jnp.isfinite(e_max), e_max, 0.0)
    ex = jnp.exp(e - e_max[dst])
    denom = jax.ops.segment_sum(ex, dst, num_segments=n)
    alpha = ex / (denom[dst] + 1e-16)
    msg = h[src] * alpha[:, :, None]
    out = jax.ops.segment_sum(msg, dst, num_segments=n)
    if concat:
        out = out.reshape(n, heads * out_ch)
    else:
        out = out.mean(axis=1)
    return out + bias


def kernel(temporal_data, x, edge_index, Wt1, bt1, Wt2, bt2, W1, as1, ad1, b1,
           W2, as2, ad2, b2, Wc1, bc1, Wc2, bc2):
    t_mean = _temporal_mlp(temporal_data, Wt1, bt1, Wt2, bt2)

    loop = jnp.arange(N, dtype=edge_index.dtype)
    src = jnp.concatenate([edge_index[0], loop])
    dst = jnp.concatenate([edge_index[1], loop])

    g = _gat_conv(x, src, dst, W1, as1, ad1, b1, 4, 32, True)
    g = jax.nn.elu(g)
    g = _gat_conv(g, src, dst, W2, as2, ad2, b2, 1, 64, False)

    return _head_mlp(t_mean, g, Wc1, bc1, Wc2, bc2)


# speedup vs baseline: 40.6019x; 40.5715x over previous
import dataclasses
import functools

import jax
import jax.numpy as jnp
from jax import lax
from jax.experimental import pallas as pl
from jax.experimental.pallas import tpu as pltpu
from jax.experimental.pallas import tpu_sc as plsc

N = 50000
T = 20
E = 1600000

NB = 5
R = 10240
NP = NB * R
NTILES = 32
WB = 2048
NW1 = 26
CH = NW1 * WB
EPAD = NTILES * CH
CAP = 53760
W = 128
SENT = 1 << 20
D1 = 144
D2 = 80
RPT = R // 16

_mesh = plsc.VectorSubcoreMesh(core_axis_name="c", subcore_axis_name="s")

_sc_params = pltpu.CompilerParams()
if "needs_layout_passes" in pltpu.CompilerParams.__dataclass_fields__:
    _sc_params = dataclasses.replace(_sc_params, needs_layout_passes=False)
if "use_tc_tiling_on_sc" in pltpu.CompilerParams.__dataclass_fields__:
    _sc_params = dataclasses.replace(_sc_params, use_tc_tiling_on_sc=False)


def _t0_body(td_ref, wt1_ref, bt1_ref, wt2_ref, bt2_ref, o_ref):
    y = jnp.maximum(td_ref[...] @ wt1_ref[...] + bt1_ref[...][None, :], 0.0)
    b = y.shape[0] // T
    ym = y.reshape(b, T, 64).mean(axis=1)
    o_ref[...] = ym @ wt2_ref[...] + bt2_ref[...][None, :]


def _temporal_mlp(td, Wt1, bt1, Wt2, bt2):
    B = 400
    td2 = td.reshape(N * T, 10)
    return pl.pallas_call(
        _t0_body,
        grid=(N // B,),
        in_specs=[
            pl.BlockSpec((B * T, 10), lambda i: (i, 0)),
            pl.BlockSpec((10, 64), lambda i: (0, 0)),
            pl.BlockSpec((64,), lambda i: (0,)),
            pl.BlockSpec((64, 64), lambda i: (0, 0)),
            pl.BlockSpec((64,), lambda i: (0,)),
        ],
        out_specs=pl.BlockSpec((B, 64), lambda i: (i, 0)),
        out_shape=jax.ShapeDtypeStruct((N, 64), jnp.float32),
    )(td2, Wt1, bt1, Wt2, bt2)


def _t1_body(x_ref, w1_ref, as1_ref, ad1_ref, hp_ref, ad_ref):
    h = x_ref[...] @ w1_ref[...]
    hp_ref[:, 0:128] = h
    hp_ref[:, 132:144] = jnp.zeros((x_ref.shape[0], 12), jnp.float32)
    ad_ref[:, 4:16] = jnp.zeros((x_ref.shape[0], 12), jnp.float32)
    for hh in range(4):
        hs = h[:, hh * 32:(hh + 1) * 32]
        hp_ref[:, 128 + hh:129 + hh] = (hs * as1_ref[hh][None, :]).sum(
            -1, keepdims=True)
        ad_ref[:, hh:hh + 1] = (hs * ad1_ref[hh][None, :]).sum(-1, keepdims=True)


def _pack1(xp, W1, as1, ad1):
    B = 1600
    return pl.pallas_call(
        _t1_body,
        grid=(NP // B,),
        in_specs=[
            pl.BlockSpec((B, 10), lambda i: (i, 0)),
            pl.BlockSpec((10, 128), lambda i: (0, 0)),
            pl.BlockSpec((4, 32), lambda i: (0, 0)),
            pl.BlockSpec((4, 32), lambda i: (0, 0)),
        ],
        out_specs=[
            pl.BlockSpec((B, D1), lambda i: (i, 0)),
            pl.BlockSpec((B, 16), lambda i: (i, 0)),
        ],
        out_shape=[
            jax.ShapeDtypeStruct((NP, D1), jnp.float32),
            jax.ShapeDtypeStruct((NP, 16), jnp.float32),
        ],
    )(xp, W1, as1, ad1)


def _t2_body(r_ref, b1_ref, w2_ref, as2_ref, ad2_ref, hp_ref, ad_ref):
    num = r_ref[0, :, 0:128] + r_ref[1, :, 0:128]
    den = r_ref[0, :, 128:132] + r_ref[1, :, 128:132]
    B = num.shape[0]
    cols = []
    for hh in range(4):
        cols.append(num[:, hh * 32:(hh + 1) * 32]
                    / (den[:, hh:hh + 1] + 1e-16))
    g = jnp.concatenate(cols, axis=1) + b1_ref[...][None, :]
    g = jnp.where(g > 0, g, jnp.exp(g) - 1.0)
    h2 = g @ w2_ref[...]
    hp_ref[:, 0:64] = h2
    hp_ref[:, 64:65] = (h2 * as2_ref[0][None, :]).sum(-1, keepdims=True)
    hp_ref[:, 65:80] = jnp.zeros((B, 15), jnp.float32)
    ad_ref[:, 0:1] = (h2 * ad2_ref[0][None, :]).sum(-1, keepdims=True)
    ad_ref[:, 1:16] = jnp.zeros((B, 15), jnp.float32)


def _finalize1_pack2(raw1, b1, W2, as2, ad2):
    B = 1600
    return pl.pallas_call(
        _t2_body,
        grid=(NP // B,),
        in_specs=[
            pl.BlockSpec((2, B, D1), lambda i: (0, i, 0)),
            pl.BlockSpec((128,), lambda i: (0,)),
            pl.BlockSpec((128, 64), lambda i: (0, 0)),
            pl.BlockSpec((1, 64), lambda i: (0, 0)),
            pl.BlockSpec((1, 64), lambda i: (0, 0)),
        ],
        out_specs=[
            pl.BlockSpec((B, D2), lambda i: (i, 0)),
            pl.BlockSpec((B, 16), lambda i: (i, 0)),
        ],
        out_shape=[
            jax.ShapeDtypeStruct((NP, D2), jnp.float32),
            jax.ShapeDtypeStruct((NP, 16), jnp.float32),
        ],
    )(raw1, b1, W2, as2, ad2)


def _t3_body(r_ref, b2_ref, t_ref, wc1_ref, bc1_ref, wc2_ref, bc2_ref, o_ref):
    num = r_ref[0, :, 0:64] + r_ref[1, :, 0:64]
    den = r_ref[0, :, 64:65] + r_ref[1, :, 64:65]
    g2 = num / (den + 1e-16) + b2_ref[...][None, :]
    c = jnp.concatenate([t_ref[...], g2], axis=1)
    h = jnp.maximum(c @ wc1_ref[...] + bc1_ref[...][None, :], 0.0)
    o_ref[...] = jax.nn.sigmoid(h @ wc2_ref[...] + bc2_ref[...][None, :])


def _finalize2_head(raw2, b2, t_mean, Wc1, bc1, Wc2, bc2):
    B = 2000
    return pl.pallas_call(
        _t3_body,
        grid=(N // B,),
        in_specs=[
            pl.BlockSpec((2, B, D2), lambda i: (0, i, 0)),
            pl.BlockSpec((64,), lambda i: (0,)),
            pl.BlockSpec((B, 64), lambda i: (i, 0)),
            pl.BlockSpec((128, 64), lambda i: (0, 0)),
            pl.BlockSpec((64,), lambda i: (0,)),
            pl.BlockSpec((64, 1), lambda i: (0, 0)),
            pl.BlockSpec((1,), lambda i: (0,)),
        ],
        out_specs=pl.BlockSpec((B, 1), lambda i: (i, 0)),
        out_shape=jax.ShapeDtypeStruct((N, 1), jnp.float32),
    )(raw2, b2, t_mean, Wc1, bc1, Wc2, bc2)


@functools.partial(
    pl.kernel,
    mesh=_mesh,
    out_type=[
        jax.ShapeDtypeStruct((NTILES * NB * CAP,), jnp.int32),
        jax.ShapeDtypeStruct((NTILES * NB * CAP,), jnp.int32),
        jax.ShapeDtypeStruct((NTILES, 16), jnp.int32),
    ],
    scratch_types=[
        pltpu.VMEM((WB,), jnp.int32),
        pltpu.VMEM((WB,), jnp.int32),
    ] + [pltpu.VMEM((WB + 16,), jnp.int32) for _ in range(2 * NB)] + [
        pltpu.VMEM((16,), jnp.int32),
    ],
    compiler_params=_sc_params,
)
def _k1_bucket(src_hbm, dst_hbm, bsrc_hbm, bdst_hbm, cnt_hbm,
               sw, dw, cs0, cs1, cs2, cs3, cs4, cd0, cd1, cd2, cd3, cd4, cv):
    csrc = [cs0, cs1, cs2, cs3, cs4]
    cdst = [cd0, cd1, cd2, cd3, cd4]
    wid = lax.axis_index("c") * 16 + lax.axis_index("s")
    base_e = wid * CH
    zero16 = jnp.zeros((16,), jnp.int32)
    sent16 = jnp.full((16,), SENT, jnp.int32)

    def window(w, g):
        off = pl.multiple_of(base_e + w * WB, 16)
        pltpu.sync_copy(src_hbm.at[pl.ds(off, WB)], sw)
        pltpu.sync_copy(dst_hbm.at[pl.ds(off, WB)], dw)

        def group(i, curs):
            s16 = sw[pl.ds(i * 16, 16)]
            d16 = dw[pl.ds(i * 16, 16)]
            new = []
            for b in range(NB):
                m = (d16 >= b * R) & (d16 < (b + 1) * R)
                plsc.store_compressed(csrc[b].at[pl.ds(curs[b], 16)], s16,
                                      mask=m)
                plsc.store_compressed(cdst[b].at[pl.ds(curs[b], 16)], d16,
                                      mask=m)
                new.append(curs[b] + jnp.sum(m.astype(jnp.int32)))
            return tuple(new)

        curs = lax.fori_loop(0, WB // 16, group,
                             (jnp.int32(0),) * NB)
        gn = []
        for b in range(NB):
            cb = curs[b]
            csrc[b][pl.ds(cb, 16)] = zero16
            cdst[b][pl.ds(cb, 16)] = sent16
            cbp = (cb + 15) & (-16)
            gaddr = pl.multiple_of((wid * NB + b) * CAP + g[b], 16)
            pltpu.sync_copy(csrc[b], bsrc_hbm.at[pl.ds(gaddr, WB + 16)])
            pltpu.sync_copy(cdst[b], bdst_hbm.at[pl.ds(gaddr, WB + 16)])
            gn.append(g[b] + cbp)
        return tuple(gn)

    gfin = lax.fori_loop(0, NW1, window, (jnp.int32(0),) * NB)
    cv[...] = jnp.zeros((16,), jnp.int32)
    for b in range(NB):
        plsc.store_scatter(cv, [jnp.full((16,), b, jnp.int32)],
                           jnp.full((16,), gfin[b], jnp.int32))
    pltpu.sync_copy(cv, cnt_hbm.at[wid])


def _make_edge_pass(D, HD, NH):
    zrows = RPT // W
    cph = HD // NH // 16

    @functools.partial(
        pl.kernel,
        mesh=_mesh,
        out_type=jax.ShapeDtypeStruct((2, NB, R, D), jnp.float32),
        scratch_types=[
            pltpu.VMEM_SHARED((R, D), jnp.float32),
            pltpu.VMEM((W, D), jnp.float32),
            pltpu.VMEM((W, 16), jnp.float32),
            pltpu.VMEM((W,), jnp.int32),
            pltpu.VMEM((W,), jnp.int32),
            pltpu.VMEM((W,), jnp.int32),
            pltpu.VMEM((W, 4), jnp.float32),
            pltpu.VMEM((W,), jnp.int32),
            pltpu.VMEM((W,), jnp.int32),
            pltpu.VMEM((16,), jnp.int32),
        ],
        compiler_params=_sc_params,
    )
    def edge_pass(hp_hbm, ad_hbm, bsrc_hbm, bdst_hbm, cnt_hbm, raw_hbm,
                  acc, rows, adv, sidx, didx, gidx, wbuf, sbuf, dbuf, cv):
        cid = lax.axis_index("c")
        sid = lax.axis_index("s")
        wid = cid * 16 + sid
        lane = lax.iota(jnp.int32, 16)
        zf16 = jnp.zeros((16,), jnp.float32)
        pltpu.sync_copy(cnt_hbm.at[wid], cv)

        for b in range(NB):
            def zrow(r, _):
                for c in range(D // 16):
                    rows[r, pl.ds(c * 16, 16)] = zf16
                return 0

            lax.fori_loop(0, W, zrow, 0)
            for z in range(zrows):
                pltpu.sync_copy(rows, acc.at[pl.ds(sid * RPT + z * W, W)])
            plsc.subcore_barrier()

            cnt = jnp.max(jnp.where(lane == b, cv[...], 0))
            nw = (cnt + (W - 1)) // W
            seg = (wid * NB + b) * CAP

            def window(i, _):
                off = pl.multiple_of(seg + i * W, 16)
                pltpu.sync_copy(bsrc_hbm.at[pl.ds(off, W)], sbuf)
                pltpu.sync_copy(bdst_hbm.at[pl.ds(off, W)], dbuf)

                def group(j, _):
                    s16 = sbuf[pl.ds(j * 16, 16)]
                    sidx[pl.ds(j * 16, 16)] = jnp.clip(s16, 0, N - 1)
                    d16 = dbuf[pl.ds(j * 16, 16)] - b * R
                    dl = jnp.clip(d16, 0, R - 1)
                    didx[pl.ds(j * 16, 16)] = dl
                    gidx[pl.ds(j * 16, 16)] = dl + b * R
                    return 0

                lax.fori_loop(0, W // 16, group, 0)
                pltpu.sync_copy(hp_hbm.at[sidx], rows)
                pltpu.sync_copy(ad_hbm.at[gidx], adv)

                def group2(j, _):
                    k16 = lax.iota(jnp.int32, 16) + j * 16
                    d16 = dbuf[pl.ds(j * 16, 16)] - b * R
                    valid = (d16 >= 0) & (d16 < R)
                    for hh in range(NH):
                        hcol = jnp.full((16,), HD + hh, jnp.int32)
                        adstv = plsc.load_gather(
                            adv, [k16, jnp.full((16,), hh, jnp.int32)])
                        asrcv = plsc.load_gather(rows, [k16, hcol])
                        e = asrcv + adstv
                        e = jnp.where(e > 0, e, 0.2 * e)
                        w = jnp.where(valid, jnp.exp(e), 0.0)
                        plsc.store_scatter(rows, [k16, hcol], w)
                        plsc.store_scatter(
                            wbuf, [k16, jnp.full((16,), hh, jnp.int32)], w)
                    return 0

                lax.fori_loop(0, W // 16, group2, 0)
                NH_stride = HD // NH

                def edge(k, _):
                    kf = jnp.full((16,), k, jnp.int32)
                    for hh in range(NH):
                        wv = plsc.load_gather(
                            wbuf, [kf, jnp.full((16,), hh, jnp.int32)])
                        for cc in range(cph):
                            col = (hh * NH_stride) + cc * 16
                            rows[k, pl.ds(col, 16)] = (
                                rows[k, pl.ds(col, 16)] * wv)
                    return 0

                lax.fori_loop(0, W, edge, 0)
                pltpu.sync_copy(rows, acc.at[didx], add=True)
                return 0

            lax.fori_loop(0, nw, window, 0)
            plsc.subcore_barrier()
            for z in range(zrows):
                pltpu.sync_copy(
                    acc.at[pl.ds(sid * RPT + z * W, W)],
                    raw_hbm.at[cid, b, pl.ds(sid * RPT + z * W, W)])
            plsc.subcore_barrier()

    return edge_pass


_edge_pass1 = _make_edge_pass(D1, 128, 4)
_edge_pass2 = _make_edge_pass(D2, 64, 1)


def kernel(temporal_data, x, edge_index, Wt1, bt1, Wt2, bt2, W1, as1, ad1, b1,
           W2, as2, ad2, b2, Wc1, bc1, Wc2, bc2):
    loop = jnp.arange(N, dtype=jnp.int32)
    npad = EPAD - E - N
    srcp = jnp.concatenate(
        [edge_index[0].astype(jnp.int32), loop, jnp.zeros((npad,), jnp.int32)])
    dstp = jnp.concatenate(
        [edge_index[1].astype(jnp.int32), loop,
         jnp.full((npad,), SENT, jnp.int32)])
    xp = jnp.pad(x, ((0, NP - N), (0, 0)))

    t_mean = _temporal_mlp(temporal_data, Wt1, bt1, Wt2, bt2)
    bsrc, bdst, counts = _k1_bucket(srcp, dstp)

    hp1, ad1p = _pack1(xp, W1, as1, ad1)
    raw1 = _edge_pass1(hp1, ad1p, bsrc, bdst, counts)
    raw1 = raw1.reshape(2, NP, D1)

    hp2, ad2p = _finalize1_pack2(raw1, b1, W2, as2, ad2)
    raw2 = _edge_pass2(hp2, ad2p, bsrc, bdst, counts)
    raw2 = raw2.reshape(2, NP, D2)

    return _finalize2_head(raw2, b2, t_mean, Wc1, bc1, Wc2, bc2)


# speedup vs baseline: 46.4830x; 1.1448x over previous
import dataclasses
import functools

import jax
import jax.numpy as jnp
from jax import lax
from jax.experimental import pallas as pl
from jax.experimental.pallas import tpu as pltpu
from jax.experimental.pallas import tpu_sc as plsc

N = 50000
T = 20
E = 1600000

NB = 5
R = 10240
NP = NB * R
NTILES = 32
WB = 2048
NW1 = 26
CH = NW1 * WB
EPAD = NTILES * CH
CAP = 53760
W = 192
SENT = 1 << 20
D1 = 144
D2 = 80
RPT = R // 16

_mesh = plsc.VectorSubcoreMesh(core_axis_name="c", subcore_axis_name="s")

_sc_params = pltpu.CompilerParams()
if "needs_layout_passes" in pltpu.CompilerParams.__dataclass_fields__:
    _sc_params = dataclasses.replace(_sc_params, needs_layout_passes=False)
if "use_tc_tiling_on_sc" in pltpu.CompilerParams.__dataclass_fields__:
    _sc_params = dataclasses.replace(_sc_params, use_tc_tiling_on_sc=False)


def _t0_body(td_ref, wt1_ref, bt1_ref, wt2_ref, bt2_ref, o_ref):
    y = jnp.maximum(td_ref[...] @ wt1_ref[...] + bt1_ref[...][None, :], 0.0)
    b = y.shape[0] // T
    ym = y.reshape(b, T, 64).mean(axis=1)
    o_ref[...] = ym @ wt2_ref[...] + bt2_ref[...][None, :]


def _temporal_mlp(td, Wt1, bt1, Wt2, bt2):
    B = 400
    td2 = td.reshape(N * T, 10)
    return pl.pallas_call(
        _t0_body,
        grid=(N // B,),
        in_specs=[
            pl.BlockSpec((B * T, 10), lambda i: (i, 0)),
            pl.BlockSpec((10, 64), lambda i: (0, 0)),
            pl.BlockSpec((64,), lambda i: (0,)),
            pl.BlockSpec((64, 64), lambda i: (0, 0)),
            pl.BlockSpec((64,), lambda i: (0,)),
        ],
        out_specs=pl.BlockSpec((B, 64), lambda i: (i, 0)),
        out_shape=jax.ShapeDtypeStruct((N, 64), jnp.float32),
    )(td2, Wt1, bt1, Wt2, bt2)


def _t1_body(x_ref, w1_ref, as1_ref, ad1_ref, hp_ref, ad_ref):
    h = x_ref[...] @ w1_ref[...]
    hp_ref[:, 0:128] = h
    hp_ref[:, 132:144] = jnp.zeros((x_ref.shape[0], 12), jnp.float32)
    ad_ref[:, 4:16] = jnp.zeros((x_ref.shape[0], 12), jnp.float32)
    for hh in range(4):
        hs = h[:, hh * 32:(hh + 1) * 32]
        hp_ref[:, 128 + hh:129 + hh] = (hs * as1_ref[hh][None, :]).sum(
            -1, keepdims=True)
        ad_ref[:, hh:hh + 1] = (hs * ad1_ref[hh][None, :]).sum(-1, keepdims=True)


def _pack1(xp, W1, as1, ad1):
    B = 1600
    return pl.pallas_call(
        _t1_body,
        grid=(NP // B,),
        in_specs=[
            pl.BlockSpec((B, 10), lambda i: (i, 0)),
            pl.BlockSpec((10, 128), lambda i: (0, 0)),
            pl.BlockSpec((4, 32), lambda i: (0, 0)),
            pl.BlockSpec((4, 32), lambda i: (0, 0)),
        ],
        out_specs=[
            pl.BlockSpec((B, D1), lambda i: (i, 0)),
            pl.BlockSpec((B, 16), lambda i: (i, 0)),
        ],
        out_shape=[
            jax.ShapeDtypeStruct((NP, D1), jnp.float32),
            jax.ShapeDtypeStruct((NP, 16), jnp.float32),
        ],
    )(xp, W1, as1, ad1)


def _t2_body(r_ref, b1_ref, w2_ref, as2_ref, ad2_ref, hp_ref, ad_ref):
    num = r_ref[0, :, 0:128] + r_ref[1, :, 0:128]
    den = r_ref[0, :, 128:132] + r_ref[1, :, 128:132]
    B = num.shape[0]
    cols = []
    for hh in range(4):
        cols.append(num[:, hh * 32:(hh + 1) * 32]
                    / (den[:, hh:hh + 1] + 1e-16))
    g = jnp.concatenate(cols, axis=1) + b1_ref[...][None, :]
    g = jnp.where(g > 0, g, jnp.exp(g) - 1.0)
    h2 = g @ w2_ref[...]
    hp_ref[:, 0:64] = h2
    hp_ref[:, 64:65] = (h2 * as2_ref[0][None, :]).sum(-1, keepdims=True)
    hp_ref[:, 65:80] = jnp.zeros((B, 15), jnp.float32)
    ad_ref[:, 0:1] = (h2 * ad2_ref[0][None, :]).sum(-1, keepdims=True)
    ad_ref[:, 1:16] = jnp.zeros((B, 15), jnp.float32)


def _finalize1_pack2(raw1, b1, W2, as2, ad2):
    B = 1600
    return pl.pallas_call(
        _t2_body,
        grid=(NP // B,),
        in_specs=[
            pl.BlockSpec((2, B, D1), lambda i: (0, i, 0)),
            pl.BlockSpec((128,), lambda i: (0,)),
            pl.BlockSpec((128, 64), lambda i: (0, 0)),
            pl.BlockSpec((1, 64), lambda i: (0, 0)),
            pl.BlockSpec((1, 64), lambda i: (0, 0)),
        ],
        out_specs=[
            pl.BlockSpec((B, D2), lambda i: (i, 0)),
            pl.BlockSpec((B, 16), lambda i: (i, 0)),
        ],
        out_shape=[
            jax.ShapeDtypeStruct((NP, D2), jnp.float32),
            jax.ShapeDtypeStruct((NP, 16), jnp.float32),
        ],
    )(raw1, b1, W2, as2, ad2)


def _t3_body(r_ref, b2_ref, t_ref, wc1_ref, bc1_ref, wc2_ref, bc2_ref, o_ref):
    num = r_ref[0, :, 0:64] + r_ref[1, :, 0:64]
    den = r_ref[0, :, 64:65] + r_ref[1, :, 64:65]
    g2 = num / (den + 1e-16) + b2_ref[...][None, :]
    c = jnp.concatenate([t_ref[...], g2], axis=1)
    h = jnp.maximum(c @ wc1_ref[...] + bc1_ref[...][None, :], 0.0)
    o_ref[...] = jax.nn.sigmoid(h @ wc2_ref[...] + bc2_ref[...][None, :])


def _finalize2_head(raw2, b2, t_mean, Wc1, bc1, Wc2, bc2):
    B = 2000
    return pl.pallas_call(
        _t3_body,
        grid=(N // B,),
        in_specs=[
            pl.BlockSpec((2, B, D2), lambda i: (0, i, 0)),
            pl.BlockSpec((64,), lambda i: (0,)),
            pl.BlockSpec((B, 64), lambda i: (i, 0)),
            pl.BlockSpec((128, 64), lambda i: (0, 0)),
            pl.BlockSpec((64,), lambda i: (0,)),
            pl.BlockSpec((64, 1), lambda i: (0, 0)),
            pl.BlockSpec((1,), lambda i: (0,)),
        ],
        out_specs=pl.BlockSpec((B, 1), lambda i: (i, 0)),
        out_shape=jax.ShapeDtypeStruct((N, 1), jnp.float32),
    )(raw2, b2, t_mean, Wc1, bc1, Wc2, bc2)


@functools.partial(
    pl.kernel,
    mesh=_mesh,
    out_type=[
        jax.ShapeDtypeStruct((NTILES * NB * CAP,), jnp.int32),
        jax.ShapeDtypeStruct((NTILES * NB * CAP,), jnp.int32),
        jax.ShapeDtypeStruct((NTILES, 16), jnp.int32),
    ],
    scratch_types=[
        pltpu.VMEM((WB,), jnp.int32),
        pltpu.VMEM((WB,), jnp.int32),
    ] + [pltpu.VMEM((WB + 16,), jnp.int32) for _ in range(2 * NB)] + [
        pltpu.VMEM((16,), jnp.int32),
    ],
    compiler_params=_sc_params,
)
def _k1_bucket(src_hbm, dst_hbm, bsrc_hbm, bdst_hbm, cnt_hbm,
               sw, dw, cs0, cs1, cs2, cs3, cs4, cd0, cd1, cd2, cd3, cd4, cv):
    csrc = [cs0, cs1, cs2, cs3, cs4]
    cdst = [cd0, cd1, cd2, cd3, cd4]
    wid = lax.axis_index("c") * 16 + lax.axis_index("s")
    base_e = wid * CH
    zero16 = jnp.zeros((16,), jnp.int32)
    sent16 = jnp.full((16,), SENT, jnp.int32)

    def window(w, g):
        off = pl.multiple_of(base_e + w * WB, 16)
        pltpu.sync_copy(src_hbm.at[pl.ds(off, WB)], sw)
        pltpu.sync_copy(dst_hbm.at[pl.ds(off, WB)], dw)

        def group(i, curs):
            s16 = sw[pl.ds(i * 16, 16)]
            d16 = dw[pl.ds(i * 16, 16)]
            new = []
            for b in range(NB):
                m = (d16 >= b * R) & (d16 < (b + 1) * R)
                plsc.store_compressed(csrc[b].at[pl.ds(curs[b], 16)], s16,
                                      mask=m)
                plsc.store_compressed(cdst[b].at[pl.ds(curs[b], 16)], d16,
                                      mask=m)
                new.append(curs[b] + jnp.sum(m.astype(jnp.int32)))
            return tuple(new)

        curs = lax.fori_loop(0, WB // 16, group,
                             (jnp.int32(0),) * NB)
        gn = []
        for b in range(NB):
            cb = curs[b]
            csrc[b][pl.ds(cb, 16)] = zero16
            cdst[b][pl.ds(cb, 16)] = sent16
            cbp = (cb + 15) & (-16)
            gaddr = pl.multiple_of((wid * NB + b) * CAP + g[b], 16)
            pltpu.sync_copy(csrc[b], bsrc_hbm.at[pl.ds(gaddr, WB + 16)])
            pltpu.sync_copy(cdst[b], bdst_hbm.at[pl.ds(gaddr, WB + 16)])
            gn.append(g[b] + cbp)
        return tuple(gn)

    gfin = lax.fori_loop(0, NW1, window, (jnp.int32(0),) * NB)
    cv[...] = jnp.zeros((16,), jnp.int32)
    for b in range(NB):
        plsc.store_scatter(cv, [jnp.full((16,), b, jnp.int32)],
                           jnp.full((16,), gfin[b], jnp.int32))
    pltpu.sync_copy(cv, cnt_hbm.at[wid])


def _make_edge_pass(D, HD, NH):
    ZW = 128
    zrows = RPT // ZW
    cph = HD // NH // 16

    @functools.partial(
        pl.kernel,
        mesh=_mesh,
        out_type=jax.ShapeDtypeStruct((2, NB, R, D), jnp.float32),
        scratch_types=[
            pltpu.VMEM_SHARED((R, D), jnp.float32),
            pltpu.VMEM((W, D), jnp.float32),
            pltpu.VMEM((W, 16), jnp.float32),
            pltpu.VMEM((W,), jnp.int32),
            pltpu.VMEM((W,), jnp.int32),
            pltpu.VMEM((W,), jnp.int32),
            pltpu.VMEM((W,), jnp.int32),
            pltpu.VMEM((W,), jnp.int32),
            pltpu.VMEM((16,), jnp.int32),
        ],
        compiler_params=_sc_params,
    )
    def edge_pass(hp_hbm, ad_hbm, bsrc_hbm, bdst_hbm, cnt_hbm, raw_hbm,
                  acc, rows, adv, sidx, didx, gidx, sbuf, dbuf, cv):
        cid = lax.axis_index("c")
        sid = lax.axis_index("s")
        wid = cid * 16 + sid
        lane = lax.iota(jnp.int32, 16)
        zf16 = jnp.zeros((16,), jnp.float32)
        pltpu.sync_copy(cnt_hbm.at[wid], cv)

        for b in range(NB):
            def zrow(r, _):
                for c in range(D // 16):
                    rows[r, pl.ds(c * 16, 16)] = zf16
                return 0

            lax.fori_loop(0, ZW, zrow, 0)
            for z in range(zrows):
                pltpu.sync_copy(rows.at[pl.ds(0, ZW)],
                                acc.at[pl.ds(sid * RPT + z * ZW, ZW)])
            plsc.subcore_barrier()

            cnt = jnp.max(jnp.where(lane == b, cv[...], 0))
            nw = (cnt + (W - 1)) // W
            seg = (wid * NB + b) * CAP

            def window(i, _):
                off = pl.multiple_of(seg + i * W, 16)
                pltpu.sync_copy(bsrc_hbm.at[pl.ds(off, W)], sbuf)
                pltpu.sync_copy(bdst_hbm.at[pl.ds(off, W)], dbuf)

                def group(j, _):
                    s16 = sbuf[pl.ds(j * 16, 16)]
                    sidx[pl.ds(j * 16, 16)] = jnp.clip(s16, 0, N - 1)
                    d16 = dbuf[pl.ds(j * 16, 16)] - b * R
                    dl = jnp.clip(d16, 0, R - 1)
                    didx[pl.ds(j * 16, 16)] = dl
                    gidx[pl.ds(j * 16, 16)] = dl + b * R
                    return 0

                lax.fori_loop(0, W // 16, group, 0)
                pltpu.sync_copy(hp_hbm.at[sidx], rows)
                pltpu.sync_copy(ad_hbm.at[gidx], adv)

                def group2(j, _):
                    k16 = lax.iota(jnp.int32, 16) + j * 16
                    d16 = dbuf[pl.ds(j * 16, 16)] - b * R
                    valid = (d16 >= 0) & (d16 < R)
                    ws = []
                    for hh in range(NH):
                        hcol = jnp.full((16,), HD + hh, jnp.int32)
                        adstv = plsc.load_gather(
                            adv, [k16, jnp.full((16,), hh, jnp.int32)])
                        asrcv = plsc.load_gather(rows, [k16, hcol])
                        e = asrcv + adstv
                        e = jnp.where(e > 0, e, 0.2 * e)
                        w = jnp.where(valid, jnp.exp(e), 0.0)
                        plsc.store_scatter(rows, [k16, hcol], w)
                        ws.append(w)
                    base = j * 16
                    for l in range(16):
                        k = base + l
                        for hh in range(NH):
                            wv = jnp.full((16,), ws[hh][l])
                            for cc in range(cph):
                                col = (hh * (HD // NH)) + cc * 16
                                rows[k, pl.ds(col, 16)] = (
                                    rows[k, pl.ds(col, 16)] * wv)
                    return 0

                lax.fori_loop(0, W // 16, group2, 0)
                pltpu.sync_copy(rows, acc.at[didx], add=True)
                return 0

            lax.fori_loop(0, nw, window, 0)
            plsc.subcore_barrier()
            for z in range(zrows):
                pltpu.sync_copy(
                    acc.at[pl.ds(sid * RPT + z * ZW, ZW)],
                    raw_hbm.at[cid, b, pl.ds(sid * RPT + z * ZW, ZW)])
            plsc.subcore_barrier()

    return edge_pass


_edge_pass1 = _make_edge_pass(D1, 128, 4)
_edge_pass2 = _make_edge_pass(D2, 64, 1)


def kernel(temporal_data, x, edge_index, Wt1, bt1, Wt2, bt2, W1, as1, ad1, b1,
           W2, as2, ad2, b2, Wc1, bc1, Wc2, bc2):
    loop = jnp.arange(N, dtype=jnp.int32)
    npad = EPAD - E - N
    srcp = jnp.concatenate(
        [edge_index[0].astype(jnp.int32), loop, jnp.zeros((npad,), jnp.int32)])
    dstp = jnp.concatenate(
        [edge_index[1].astype(jnp.int32), loop,
         jnp.full((npad,), SENT, jnp.int32)])
    xp = jnp.pad(x, ((0, NP - N), (0, 0)))

    t_mean = _temporal_mlp(temporal_data, Wt1, bt1, Wt2, bt2)
    bsrc, bdst, counts = _k1_bucket(srcp, dstp)

    hp1, ad1p = _pack1(xp, W1, as1, ad1)
    raw1 = _edge_pass1(hp1, ad1p, bsrc, bdst, counts)
    raw1 = raw1.reshape(2, NP, D1)

    hp2, ad2p = _finalize1_pack2(raw1, b1, W2, as2, ad2)
    raw2 = _edge_pass2(hp2, ad2p, bsrc, bdst, counts)
    raw2 = raw2.reshape(2, NP, D2)

    return _finalize2_head(raw2, b2, t_mean, Wc1, bc1, Wc2, bc2)
